# Initial kernel scaffold; baseline (speedup 1.0000x reference)
import jax
import jax.numpy as jnp
from jax.experimental import pallas as pl


def kernel(inputs, embedding_weight):
    raise NotImplementedError("write your pallas kernel here")


# speedup vs baseline: 1.5424x; 1.5424x over previous
import jax
import jax.numpy as jnp
from jax.experimental import pallas as pl
from jax.experimental.pallas import tpu as pltpu

NUM_EMB = 1024
DIM = 64
COMMIT = 0.25
N_ROWS = 16384
BR = 1024
NBLK = N_ROWS // BR


def _vq_body(x_ref, e_ref, enc_ref, q_ref, loss_ref, perp_ref,
             counts_ref, loss_acc):
    i = pl.program_id(0)
    x = x_ref[...]
    e = e_ref[...]
    esq = jnp.sum(e * e, axis=1)
    xsq = jnp.sum(x * x, axis=1, keepdims=True)
    xe = jax.lax.dot_general(x, e, (((1,), (1,)), ((), ())),
                             preferred_element_type=jnp.float32)
    d = xsq + esq[None, :] - 2.0 * xe
    idx = jnp.argmin(d, axis=1)
    lane = jax.lax.broadcasted_iota(jnp.int32, (BR, NUM_EMB), 1)
    enc = (lane == idx[:, None]).astype(jnp.float32)
    enc_ref[...] = enc
    q = jax.lax.dot_general(enc, e, (((1,), (0,)), ((), ())),
                            preferred_element_type=jnp.float32)
    q_ref[...] = q
    diff = q - x
    part_loss = jnp.sum(diff * diff)
    part_counts = jnp.sum(enc, axis=0, keepdims=True)

    @pl.when(i == 0)
    def _():
        loss_acc[0] = part_loss
        counts_ref[...] = part_counts

    @pl.when(i > 0)
    def _():
        loss_acc[0] += part_loss
        counts_ref[...] += part_counts

    @pl.when(i == NBLK - 1)
    def _():
        loss_ref[0, 0] = loss_acc[0] * (COMMIT / (N_ROWS * DIM))
        probs = counts_ref[...] * (1.0 / N_ROWS)
        ent = -jnp.sum(probs * jnp.log(probs + 1e-10))
        perp_ref[0, 0] = jnp.exp(ent)


def kernel(inputs, embedding_weight):
    B, C, H, W = inputs.shape
    x = jnp.transpose(inputs, (0, 2, 3, 1)).reshape(-1, C)
    enc, q, loss, perp = pl.pallas_call(
        _vq_body,
        grid=(NBLK,),
        in_specs=[
            pl.BlockSpec((BR, DIM), lambda i: (i, 0)),
            pl.BlockSpec((NUM_EMB, DIM), lambda i: (0, 0)),
        ],
        out_specs=[
            pl.BlockSpec((BR, NUM_EMB), lambda i: (i, 0)),
            pl.BlockSpec((BR, DIM), lambda i: (i, 0)),
            pl.BlockSpec(memory_space=pltpu.SMEM),
            pl.BlockSpec(memory_space=pltpu.SMEM),
        ],
        out_shape=[
            jax.ShapeDtypeStruct((N_ROWS, NUM_EMB), jnp.float32),
            jax.ShapeDtypeStruct((N_ROWS, DIM), jnp.float32),
            jax.ShapeDtypeStruct((1, 1), jnp.float32),
            jax.ShapeDtypeStruct((1, 1), jnp.float32),
        ],
        scratch_shapes=[
            pltpu.VMEM((1, NUM_EMB), jnp.float32),
            pltpu.SMEM((1,), jnp.float32),
        ],
        compiler_params=pltpu.CompilerParams(
            dimension_semantics=("arbitrary",)),
    )(x, embedding_weight)
    q_out = jnp.transpose(q.reshape(B, H, W, C), (0, 3, 1, 2))
    return loss[0, 0], q_out, perp[0, 0], enc
